# baseline (speedup 1.0000x reference)
import jax
import jax.numpy as jnp
from jax import lax
from jax.experimental import pallas as pl
from jax.experimental.pallas import tpu as pltpu
from jax.experimental.pallas import tpu_sc as plsc

VOCAB = 1000000
D = 64
NC = 2
NS = 16
NW = NC * NS

IDX_MINOR = 128
SUB_PER_CHUNK = 4
CHUNK = IDX_MINOR * SUB_PER_CHUNK


def _make_kernel(n_total: int):
    assert n_total % (NW * CHUNK) == 0
    per_w = n_total // NW
    chunks_per_w = per_w // CHUNK
    idx_rows_per_w = per_w // IDX_MINOR

    mesh = plsc.VectorSubcoreMesh(core_axis_name="c", subcore_axis_name="s")

    @pl.kernel(
        out_type=jax.ShapeDtypeStruct((n_total, D), jnp.float32),
        mesh=mesh,
        compiler_params=pltpu.CompilerParams(use_tc_tiling_on_sc=False),
        scratch_types=[
            pltpu.VMEM((SUB_PER_CHUNK, IDX_MINOR), jnp.int32),
            pltpu.VMEM((CHUNK, D), jnp.float32),
            pltpu.SemaphoreType.DMA,
            pltpu.SemaphoreType.DMA,
        ],
    )
    def emb_kernel(idx_hbm, table_hbm, out_hbm, idx_v, rows_v, gsem, ssem):
        wid = lax.axis_index("s") * NC + lax.axis_index("c")
        idx_row_base = wid * idx_rows_per_w
        row_base = wid * per_w

        @pl.loop(0, chunks_per_w)
        def chunk_loop(i):
            pltpu.sync_copy(
                idx_hbm.at[pl.ds(idx_row_base + i * SUB_PER_CHUNK, SUB_PER_CHUNK)],
                idx_v,
            )
            cps = [
                pltpu.async_copy(
                    table_hbm.at[idx_v.at[j]],
                    rows_v.at[pl.ds(j * IDX_MINOR, IDX_MINOR)],
                    gsem,
                )
                for j in range(SUB_PER_CHUNK)
            ]
            for cp in cps:
                cp.wait()
            pltpu.async_copy(
                rows_v, out_hbm.at[pl.ds(row_base + i * CHUNK, CHUNK)], ssem
            ).wait()

    return emb_kernel


def kernel(x, E):
    b, h = x.shape
    n_total = b * h
    xf = x.reshape(n_total // IDX_MINOR, IDX_MINOR).astype(jnp.int32)
    out = _make_kernel(n_total)(xf, E)
    return out.reshape(b, h, D)


# speedup vs baseline: 1.0531x; 1.0531x over previous
import jax
import jax.numpy as jnp
from jax import lax
from jax.experimental import pallas as pl
from jax.experimental.pallas import tpu as pltpu
from jax.experimental.pallas import tpu_sc as plsc

VOCAB = 1000000
D = 64
NC = 2
NS = 16
NW = NC * NS

IDX_MINOR = 128
SUB_PER_CHUNK = 4
CHUNK = IDX_MINOR * SUB_PER_CHUNK


def _make_kernel(n_total: int):
    assert n_total % (NW * CHUNK) == 0
    per_w = n_total // NW
    chunks_per_w = per_w // CHUNK
    assert chunks_per_w % 2 == 0 and chunks_per_w >= 4
    idx_rows_per_w = per_w // IDX_MINOR

    mesh = plsc.VectorSubcoreMesh(core_axis_name="c", subcore_axis_name="s")

    @pl.kernel(
        out_type=jax.ShapeDtypeStruct((n_total, D), jnp.float32),
        mesh=mesh,
        compiler_params=pltpu.CompilerParams(use_tc_tiling_on_sc=False),
        scratch_types=[
            pltpu.VMEM((SUB_PER_CHUNK, IDX_MINOR), jnp.int32),
            pltpu.VMEM((SUB_PER_CHUNK, IDX_MINOR), jnp.int32),
            pltpu.VMEM((CHUNK, D), jnp.float32),
            pltpu.VMEM((CHUNK, D), jnp.float32),
            pltpu.SemaphoreType.DMA,
            pltpu.SemaphoreType.DMA,
            pltpu.SemaphoreType.DMA,
            pltpu.SemaphoreType.DMA,
        ],
    )
    def emb_kernel(idx_hbm, table_hbm, out_hbm, idx0, idx1, rows0, rows1,
                   gsem0, gsem1, ssem0, ssem1):
        wid = lax.axis_index("s") * NC + lax.axis_index("c")
        idx_row_base = wid * idx_rows_per_w
        row_base = wid * per_w

        idx_v = (idx0, idx1)
        rows_v = (rows0, rows1)
        gsem = (gsem0, gsem1)
        ssem = (ssem0, ssem1)

        def load_idx(b, i):
            pltpu.sync_copy(
                idx_hbm.at[pl.ds(idx_row_base + i * SUB_PER_CHUNK, SUB_PER_CHUNK)],
                idx_v[b],
            )

        def fire_gathers(b):
            for j in range(SUB_PER_CHUNK):
                pltpu.async_copy(
                    table_hbm.at[idx_v[b].at[j]],
                    rows_v[b].at[pl.ds(j * IDX_MINOR, IDX_MINOR)],
                    gsem[b],
                )

        def wait_gathers(b):
            for j in range(SUB_PER_CHUNK):
                pltpu.make_async_copy(
                    table_hbm.at[idx_v[b].at[j]],
                    rows_v[b].at[pl.ds(j * IDX_MINOR, IDX_MINOR)],
                    gsem[b],
                ).wait()

        def fire_store(b, i):
            pltpu.async_copy(
                rows_v[b], out_hbm.at[pl.ds(row_base + i * CHUNK, CHUNK)], ssem[b]
            )

        def wait_store(b, i):
            pltpu.make_async_copy(
                rows_v[b], out_hbm.at[pl.ds(row_base + i * CHUNK, CHUNK)], ssem[b]
            ).wait()

        load_idx(0, 0)
        fire_gathers(0)
        load_idx(1, 1)
        fire_gathers(1)
        wait_gathers(0)
        fire_store(0, 0)

        @pl.loop(1, chunks_per_w // 2)
        def pair_loop(p):
            i0 = 2 * p
            for bb in range(2):
                i = i0 + bb
                wait_store(bb, i - 2)
                load_idx(bb, i)
                fire_gathers(bb)
                wait_gathers(1 - bb)
                fire_store(1 - bb, i - 1)

        last = chunks_per_w - 1
        wait_gathers(1)
        fire_store(1, last)
        wait_store(0, last - 1)
        wait_store(1, last)

    return emb_kernel


def kernel(x, E):
    b, h = x.shape
    n_total = b * h
    xf = x.reshape(n_total // IDX_MINOR, IDX_MINOR).astype(jnp.int32)
    out = _make_kernel(n_total)(xf, E)
    return out.reshape(b, h, D)
